# baseline (speedup 1.0000x reference)
import functools

import jax
import jax.numpy as jnp
from jax import lax
from jax.experimental import pallas as pl
from jax.experimental.pallas import tpu as pltpu
from jax.experimental.pallas import tpu_sc as plsc

N_TOK = 16384
HALF = 16
OUT_DIM = 2 * HALF
WORD_VOCAB = 1000000
SUFF_VOCAB = 1000
PACK = 8
WIDE_ROWS = WORD_VOCAB // PACK
NUM_CORES = 2
NUM_SUBCORES = 16
NW = NUM_CORES * NUM_SUBCORES
BPW = N_TOK // NW
CHUNK = 128
NCH = BPW // CHUNK
LANES = 16
NGRP = BPW // LANES

_mesh = plsc.VectorSubcoreMesh(core_axis_name="c", subcore_axis_name="s")


@functools.partial(
    pl.kernel,
    mesh=_mesh,
    out_type=jax.ShapeDtypeStruct((N_TOK * OUT_DIM,), jnp.float32),
    compiler_params=pltpu.CompilerParams(needs_layout_passes=False),
    scratch_types=[
        pltpu.VMEM((BPW,), jnp.int32),
        pltpu.VMEM((BPW,), jnp.int32),
        pltpu.VMEM((BPW,), jnp.int32),
        pltpu.VMEM((BPW, PACK * HALF), jnp.float32),
        pltpu.VMEM((SUFF_VOCAB * HALF,), jnp.float32),
        pltpu.VMEM((BPW * OUT_DIM,), jnp.float32),
        pltpu.SemaphoreType.DMA,
        pltpu.SemaphoreType.DMA,
    ],
)
def _emb_lookup(word_idx_hbm, suff_idx_hbm, W_wide_hbm, W_suff_hbm, out_hbm,
                widx_v, sidx_v, gidx_v, wrows_v, suff_v, obuf_v, gsem, ssem):
    wid = lax.axis_index("s") * NUM_CORES + lax.axis_index("c")
    base = wid * BPW

    suff_cp = pltpu.async_copy(W_suff_hbm, suff_v, ssem)

    pltpu.sync_copy(word_idx_hbm.at[pl.ds(base, BPW)], widx_v)
    pltpu.sync_copy(suff_idx_hbm.at[pl.ds(base, BPW)], sidx_v)

    for k in range(NGRP):
        gidx_v[pl.ds(k * LANES, LANES)] = (
            widx_v[pl.ds(k * LANES, LANES)] >> 3)

    copies = [
        pltpu.async_copy(
            W_wide_hbm.at[gidx_v.at[pl.ds(j * CHUNK, CHUNK)]],
            wrows_v.at[pl.ds(j * CHUNK, CHUNK)], gsem)
        for j in range(NCH)
    ]
    for c in copies:
        c.wait()
    suff_cp.wait()

    lane = lax.iota(jnp.int32, LANES)

    def group_body(g, carry):
        t_vec = g * LANES + lane
        w_vec = widx_v[pl.ds(g * LANES, LANES)]
        s_vec = sidx_v[pl.ds(g * LANES, LANES)]
        wcol = (w_vec & 7) * HALF
        sflat = s_vec * HALF
        oflat = t_vec * OUT_DIM
        for d in range(HALF):
            wv = plsc.load_gather(wrows_v, [t_vec, wcol + d])
            plsc.store_scatter(obuf_v, [oflat + d], wv)
            sv = plsc.load_gather(suff_v, [sflat + d])
            plsc.store_scatter(obuf_v, [oflat + (HALF + d)], sv)
        return carry

    lax.fori_loop(0, NGRP, group_body, 0)

    pltpu.sync_copy(obuf_v, out_hbm.at[pl.ds(base * OUT_DIM, BPW * OUT_DIM)])


def kernel(word_idx, suff_idx, W_word, W_suff):
    W_wide = W_word.reshape(WIDE_ROWS, PACK * HALF)
    W_suff_flat = W_suff.reshape(SUFF_VOCAB * HALF)
    out_flat = _emb_lookup(word_idx.astype(jnp.int32),
                           suff_idx.astype(jnp.int32),
                           W_wide, W_suff_flat)
    return out_flat.reshape(N_TOK, OUT_DIM)


# speedup vs baseline: 5.2940x; 5.2940x over previous
import functools

import jax
import jax.numpy as jnp
from jax import lax
from jax.experimental import pallas as pl
from jax.experimental.pallas import tpu as pltpu
from jax.experimental.pallas import tpu_sc as plsc

N_TOK = 16384
HALF = 16
OUT_DIM = 2 * HALF
WORD_VOCAB = 1000000
SUFF_VOCAB = 1000
NUM_CORES = 2
NUM_SUBCORES = 16
NW = NUM_CORES * NUM_SUBCORES
BPW = N_TOK // NW
CHUNK = 128
LANES = 16
NBUF = 16
NGRP = BPW // NBUF

_mesh = plsc.VectorSubcoreMesh(core_axis_name="c", subcore_axis_name="s")


@functools.partial(
    pl.kernel,
    mesh=_mesh,
    out_type=jax.ShapeDtypeStruct((OUT_DIM, N_TOK), jnp.float32),
    compiler_params=pltpu.CompilerParams(needs_layout_passes=False),
    scratch_types=[
        pltpu.VMEM((BPW,), jnp.int32),
        pltpu.VMEM((BPW,), jnp.int32),
        pltpu.VMEM((HALF, SUFF_VOCAB), jnp.float32),
        pltpu.VMEM((NBUF, HALF, CHUNK), jnp.float32),
        pltpu.VMEM((OUT_DIM, BPW), jnp.float32),
        [pltpu.SemaphoreType.DMA] * NBUF,
        pltpu.SemaphoreType.DMA,
    ],
)
def _emb_lookup(word_idx_hbm, suff_idx_hbm, WT_hbm, WST_hbm, out_hbm,
                widx_v, sidx_v, wst_v, blk_v, obuf_v, bsems, ssem):
    wid = lax.axis_index("s") * NUM_CORES + lax.axis_index("c")
    base = wid * BPW

    suff_cp = pltpu.async_copy(WST_hbm, wst_v, ssem)
    pltpu.sync_copy(word_idx_hbm.at[pl.ds(base, BPW)], widx_v)
    pltpu.sync_copy(suff_idx_hbm.at[pl.ds(base, BPW)], sidx_v)

    def fire(w_vec, b):
        col = pl.multiple_of((w_vec[b] >> 7) * CHUNK, CHUNK)
        pltpu.async_copy(WT_hbm.at[:, pl.ds(col, CHUNK)], blk_v.at[b],
                         bsems[b])

    w0 = widx_v[pl.ds(0, LANES)]
    for b in range(NBUF):
        fire(w0, b)

    suff_cp.wait()
    lane = lax.iota(jnp.int32, LANES)

    def outer(q, carry):
        s_vec = sidx_v[pl.ds(q * LANES, LANES)]
        for d in range(HALF):
            sv = plsc.load_gather(wst_v, [jnp.full((LANES,), d, jnp.int32),
                                          s_vec])
            obuf_v[HALF + d, pl.ds(q * LANES, LANES)] = sv

        w_vec = widx_v[pl.ds(q * LANES, LANES)]
        qn = jnp.minimum(q + 1, NGRP - 1)
        w_next = widx_v[pl.ds(qn * LANES, LANES)]
        for b in range(NBUF):
            t = q * NBUF + b
            pltpu.make_async_copy(WT_hbm.at[:, pl.ds(0, CHUNK)], blk_v.at[b],
                                  bsems[b]).wait()
            rem = jnp.broadcast_to(w_vec[b] & (CHUNK - 1), (LANES,))
            wv = plsc.load_gather(blk_v.at[b], [lane, rem])
            plsc.store_scatter(obuf_v,
                               [lane, jnp.full((LANES,), 0, jnp.int32) + t],
                               wv)

            @pl.when(q + 1 < NGRP)
            def _():
                fire(w_next, b)
        return carry

    lax.fori_loop(0, NGRP, outer, 0)

    pltpu.sync_copy(obuf_v, out_hbm.at[:, pl.ds(base, BPW)])


def kernel(word_idx, suff_idx, W_word, W_suff):
    out_fm = _emb_lookup(word_idx.astype(jnp.int32),
                         suff_idx.astype(jnp.int32),
                         W_word.T, W_suff.T)
    return out_fm.T
